# baseline (speedup 1.0000x reference)
import functools

import jax
import jax.numpy as jnp
from jax import lax
from jax.experimental import pallas as pl
from jax.experimental.pallas import tpu as pltpu
from jax.experimental.pallas import tpu_sc as plsc

N = 50000
E = 800000
HID = 64
HALF = 32
TDIM = 64

NC = 2
NS = 16
CH = 128
SUP = 8
CSUB = 392
NCHUNK = CSUB * NS
EP = NCHUNK * CH
NSUP = CSUB // SUP

NA = N + 8
RS = 3128
RS_LAST_A = NA - 15 * RS
RS_LAST_O = N - 15 * RS

NDEG = NCHUNK * CH // 16
DZ = 3200
DZ_LAST = NDEG - 15 * DZ

BN = 2000
GRID = N // BN


def _make_seg(with_deg):
  mesh = plsc.VectorSubcoreMesh(
      core_axis_name="c", subcore_axis_name="s",
      num_cores=NC, num_subcores=NS)

  out_type = [jax.ShapeDtypeStruct((N, HALF), jnp.float32),
              jax.ShapeDtypeStruct((N, HALF), jnp.float32)]
  scratch = [
      pltpu.VMEM_SHARED((NA, HALF), jnp.float32),
      pltpu.VMEM((SUP, CH), jnp.int32),
      pltpu.VMEM((SUP, CH), jnp.int32),
      pltpu.VMEM((SUP // 2, CH, HALF), jnp.float32),
      pltpu.SemaphoreType.DMA,
      pltpu.SemaphoreType.DMA,
  ]
  if with_deg:
    out_type.append(jax.ShapeDtypeStruct((NDEG,), jnp.float32))
    scratch += [
        pltpu.VMEM_SHARED((NDEG,), jnp.float32),
        pltpu.VMEM((CH,), jnp.float32),
    ]

  def body(*refs):
    if with_deg:
      (hlo, hhi, src2d, dst2d, zrows, zflat, ones_hbm,
       slo, shi, deg_hbm, acc, idxs, idxd, rows, gsem, ssem, dacc, ones) = refs
    else:
      (hlo, hhi, src2d, dst2d, zrows, zflat, ones_hbm,
       slo, shi, acc, idxs, idxd, rows, gsem, ssem) = refs

    c = lax.axis_index("c")
    s = lax.axis_index("s")
    r0 = s * RS

    @pl.when(s < 15)
    def _():
      pltpu.sync_copy(zrows, acc.at[pl.ds(r0, RS), :])

    @pl.when(s == 15)
    def _():
      pltpu.sync_copy(zrows.at[pl.ds(0, RS_LAST_A), :],
                      acc.at[pl.ds(15 * RS, RS_LAST_A), :])

    if with_deg:
      @pl.when(c == 0)
      def _():
        pltpu.sync_copy(ones_hbm, ones)

        @pl.when(s < 15)
        def _():
          pltpu.sync_copy(zflat.at[pl.ds(0, DZ)], dacc.at[pl.ds(s * DZ, DZ)])

        @pl.when(s == 15)
        def _():
          pltpu.sync_copy(zflat.at[pl.ds(0, DZ_LAST)],
                          dacc.at[pl.ds(15 * DZ, DZ_LAST)])
    plsc.subcore_barrier()

    def sup_body(t, carry):
      cid0 = s * CSUB + t * SUP
      pltpu.sync_copy(src2d.at[pl.ds(cid0, SUP), :], idxs)
      pltpu.sync_copy(dst2d.at[pl.ds(cid0, SUP), :], idxd)
      def fire_gather(b):
        for j in range(2):
          ch, r = 2 * b + j, 2 * (b % 2) + j

          @pl.when(c == 0)
          def _():
            pltpu.async_copy(hlo.at[idxs.at[ch]], rows.at[r], gsem)

          @pl.when(c == 1)
          def _():
            pltpu.async_copy(hhi.at[idxs.at[ch]], rows.at[r], gsem)

      def fire_scatter(b):
        for j in range(2):
          ch, r = 2 * b + j, 2 * (b % 2) + j
          pltpu.make_async_copy(hlo.at[idxs.at[ch]], rows.at[r], gsem).wait()
          pltpu.async_copy(rows.at[r], acc.at[idxd.at[ch]], ssem, add=True)
          if with_deg:
            @pl.when(c == 0)
            def _():
              pltpu.sync_copy(ones, dacc.at[idxd.at[ch]], add=True)

      def drain_scatter(b):
        for j in range(2):
          r = 2 * (b % 2) + j
          pltpu.make_async_copy(hlo.at[idxs.at[2 * b + j]], rows.at[r],
                                ssem).wait()

      fire_gather(0)
      fire_gather(1)
      fire_scatter(0)
      fire_scatter(1)
      drain_scatter(0)
      fire_gather(2)
      fire_scatter(2)
      drain_scatter(1)
      fire_gather(3)
      fire_scatter(3)
      drain_scatter(2)
      drain_scatter(3)
      return carry

    lax.fori_loop(0, NSUP, sup_body, 0)
    plsc.subcore_barrier()

    @pl.when(c == 0)
    def _():
      @pl.when(s < 15)
      def _():
        pltpu.sync_copy(acc.at[pl.ds(r0, RS), :], slo.at[pl.ds(r0, RS), :])

      @pl.when(s == 15)
      def _():
        pltpu.sync_copy(acc.at[pl.ds(15 * RS, RS_LAST_O), :],
                        slo.at[pl.ds(15 * RS, RS_LAST_O), :])

    @pl.when(c == 1)
    def _():
      @pl.when(s < 15)
      def _():
        pltpu.sync_copy(acc.at[pl.ds(r0, RS), :], shi.at[pl.ds(r0, RS), :])

      @pl.when(s == 15)
      def _():
        pltpu.sync_copy(acc.at[pl.ds(15 * RS, RS_LAST_O), :],
                        shi.at[pl.ds(15 * RS, RS_LAST_O), :])

    if with_deg:
      @pl.when((c == 0) & (s == 0))
      def _():
        pltpu.sync_copy(dacc, deg_hbm)

  return pl.kernel(body, out_type=out_type, mesh=mesh, scratch_types=scratch,
                   compiler_params=pltpu.CompilerParams(use_tc_tiling_on_sc=False),
                   name="seg_sum_deg" if with_deg else "seg_sum")


_seg_deg = _make_seg(True)
_seg = _make_seg(False)


def _full(shape):
  return pl.BlockSpec(shape, lambda i: tuple(0 for _ in shape))


def _rows(width):
  return pl.BlockSpec((BN, width), lambda i: (i, 0))


def _temb_body(t, wt1, bt1, wt2, bt2, wtime, bl, otb):
  tt = t[0, 0]
  idx = lax.broadcasted_iota(jnp.int32, (1, HALF), 1).astype(jnp.float32)
  freqs = jnp.exp(-jnp.log(10000.0) * idx / HALF)
  args = tt * freqs
  emb = jnp.concatenate([jnp.sin(args), jnp.cos(args)], axis=1)
  u = jnp.dot(emb, wt1[...], preferred_element_type=jnp.float32) + bt1[...]
  u = u * jax.nn.sigmoid(u)
  temb = jnp.dot(u, wt2[...], preferred_element_type=jnp.float32) + bt2[...]
  rows = [jnp.dot(temb, wtime[l], preferred_element_type=jnp.float32)
          + bl[l:l + 1, :] for l in range(4)]
  otb[...] = jnp.concatenate(rows, axis=0)


def _proj_body(lat, grad, w, b, olo, ohi):
  x = jnp.concatenate([lat[...], grad[...]], axis=1)
  h = jnp.dot(x, w[...], preferred_element_type=jnp.float32) + b[...]
  olo[...] = h[:, :HALF]
  ohi[...] = h[:, HALF:]


def _make_mid_body(with_skip):
  def body(*refs):
    if with_skip:
      hlo, hhi, slo, shi, deg, tb, wcat, sklo, skhi, olo, ohi = refs
    else:
      hlo, hhi, slo, shi, deg, tb, wcat, olo, ohi = refs
    h = jnp.concatenate([hlo[...], hhi[...]], axis=1)
    sn = jnp.concatenate([slo[...], shi[...]], axis=1)
    sn = sn / jnp.maximum(deg[...], 1.0)
    hs = jnp.concatenate([h, sn], axis=1)
    pre = jnp.dot(hs, wcat[...], preferred_element_type=jnp.float32) + tb[...]
    hn = jnp.maximum(pre, 0.0)
    if with_skip:
      hn = hn + jnp.concatenate([sklo[...], skhi[...]], axis=1)
    olo[...] = hn[:, :HALF]
    ohi[...] = hn[:, HALF:]
  return body


def _final_body(hlo, hhi, slo, shi, deg, tb, wcat, sklo, skhi,
                grad, wout, bout, out):
  h = jnp.concatenate([hlo[...], hhi[...]], axis=1)
  sn = jnp.concatenate([slo[...], shi[...]], axis=1)
  sn = sn / jnp.maximum(deg[...], 1.0)
  hs = jnp.concatenate([h, sn], axis=1)
  pre = jnp.dot(hs, wcat[...], preferred_element_type=jnp.float32) + tb[...]
  hn = jnp.maximum(pre, 0.0)
  hn = hn + jnp.concatenate([sklo[...], skhi[...]], axis=1)
  o = jnp.dot(hn, wout[...], preferred_element_type=jnp.float32) + bout[...]
  sg = jax.nn.sigmoid(o)
  out[...] = -grad[...] * sg[:, :HALF] + (sg[:, HALF:] * 2.0 - 1.0)


_half_out = [jax.ShapeDtypeStruct((N, HALF), jnp.float32)] * 2

_temb_call = pl.pallas_call(
    _temb_body,
    grid=(1,),
    in_specs=[_full((1, 1)), _full((TDIM, TDIM)), _full((1, TDIM)),
              _full((TDIM, HID)), _full((1, HID)), _full((4, HID, HID)),
              _full((4, HID))],
    out_specs=_full((4, HID)),
    out_shape=jax.ShapeDtypeStruct((4, HID), jnp.float32),
)

_proj_call = pl.pallas_call(
    _proj_body,
    grid=(GRID,),
    in_specs=[_rows(HALF), _rows(HALF), _full((HID, HID)), _full((1, HID))],
    out_specs=[_rows(HALF)] * 2,
    out_shape=_half_out,
)

_mid_specs = [_rows(HALF)] * 4 + [_rows(1), _full((1, HID)),
                                  _full((2 * HID, HID))]

_mid_call = pl.pallas_call(
    _make_mid_body(False),
    grid=(GRID,),
    in_specs=_mid_specs,
    out_specs=[_rows(HALF)] * 2,
    out_shape=_half_out,
)

_mid_skip_call = pl.pallas_call(
    _make_mid_body(True),
    grid=(GRID,),
    in_specs=_mid_specs + [_rows(HALF)] * 2,
    out_specs=[_rows(HALF)] * 2,
    out_shape=_half_out,
)

_final_call = pl.pallas_call(
    _final_body,
    grid=(GRID,),
    in_specs=_mid_specs + [_rows(HALF)] * 2
    + [_rows(HALF), _full((HID, HID)), _full((1, HID))],
    out_specs=_rows(HALF),
    out_shape=jax.ShapeDtypeStruct((N, HALF), jnp.float32),
)


def kernel(bxyz, latent, grad_latent, timestamp, edge_index, W_in, b_in,
           Wt1, bt1, Wt2, bt2, Wself, Wnbr, Wtime, bl, W_out, b_out):
  ei = edge_index.astype(jnp.int32)
  src2d = jnp.concatenate(
      [ei[0], jnp.zeros((EP - E,), jnp.int32)]).reshape(NCHUNK, CH)
  dst2d = jnp.concatenate(
      [ei[1], jnp.full((EP - E,), N, jnp.int32)]).reshape(NCHUNK, CH)
  zrows = jnp.zeros((RS, HALF), jnp.float32)
  zflat = jnp.zeros((DZ,), jnp.float32)
  ones_hbm = jnp.ones((CH,), jnp.float32)

  tb = _temb_call(timestamp.reshape(1, 1), Wt1, bt1.reshape(1, TDIM),
                  Wt2, bt2.reshape(1, HID), Wtime, bl)
  wcat = [jnp.concatenate([Wself[l], Wnbr[l]], axis=0) for l in range(4)]

  hlo, hhi = _proj_call(latent, grad_latent, W_in, b_in.reshape(1, HID))

  s0lo, s0hi, deg = _seg_deg(hlo, hhi, src2d, dst2d, zrows, zflat, ones_hbm)
  deg2d = deg[:N].reshape(N, 1)

  h1lo, h1hi = _mid_call(hlo, hhi, s0lo, s0hi, deg2d, tb[0:1], wcat[0])
  s1lo, s1hi = _seg(h1lo, h1hi, src2d, dst2d, zrows, zflat, ones_hbm)

  h2lo, h2hi = _mid_call(h1lo, h1hi, s1lo, s1hi, deg2d, tb[1:2], wcat[1])
  s2lo, s2hi = _seg(h2lo, h2hi, src2d, dst2d, zrows, zflat, ones_hbm)

  h3lo, h3hi = _mid_skip_call(h2lo, h2hi, s2lo, s2hi, deg2d, tb[2:3], wcat[2],
                              h2lo, h2hi)
  s3lo, s3hi = _seg(h3lo, h3hi, src2d, dst2d, zrows, zflat, ones_hbm)

  output = _final_call(h3lo, h3hi, s3lo, s3hi, deg2d, tb[3:4], wcat[3],
                       h1lo, h1hi, grad_latent, W_out, b_out.reshape(1, HID))
  return output, bxyz


# speedup vs baseline: 1.0141x; 1.0141x over previous
import functools

import jax
import jax.numpy as jnp
from jax import lax
from jax.experimental import pallas as pl
from jax.experimental.pallas import tpu as pltpu
from jax.experimental.pallas import tpu_sc as plsc

N = 50000
E = 800000
HID = 64
HALF = 32
TDIM = 64

NC = 2
NS = 16
CH = 256
SUP = 4
CSUB = 196
NCHUNK = CSUB * NS
EP = NCHUNK * CH
NSUP = CSUB // SUP

NA = N + 8
RS = 3128
RS_LAST_A = NA - 15 * RS
RS_LAST_O = N - 15 * RS

NDEG = NCHUNK * CH // 16
DZ = 3200
DZ_LAST = NDEG - 15 * DZ

BN = 2000
GRID = N // BN


def _make_seg(with_deg):
  mesh = plsc.VectorSubcoreMesh(
      core_axis_name="c", subcore_axis_name="s",
      num_cores=NC, num_subcores=NS)

  out_type = [jax.ShapeDtypeStruct((N, HALF), jnp.float32),
              jax.ShapeDtypeStruct((N, HALF), jnp.float32)]
  scratch = [
      pltpu.VMEM_SHARED((NA, HALF), jnp.float32),
      pltpu.VMEM((SUP, CH), jnp.int32),
      pltpu.VMEM((SUP, CH), jnp.int32),
      pltpu.VMEM((2, CH, HALF), jnp.float32),
      pltpu.SemaphoreType.DMA,
      pltpu.SemaphoreType.DMA,
  ]
  if with_deg:
    out_type.append(jax.ShapeDtypeStruct((NDEG,), jnp.float32))
    scratch += [
        pltpu.VMEM_SHARED((NDEG,), jnp.float32),
        pltpu.VMEM((CH,), jnp.float32),
    ]

  def body(*refs):
    if with_deg:
      (hlo, hhi, src2d, dst2d, zrows, zflat, ones_hbm,
       slo, shi, deg_hbm, acc, idxs, idxd, rows, gsem, ssem, dacc, ones) = refs
    else:
      (hlo, hhi, src2d, dst2d, zrows, zflat, ones_hbm,
       slo, shi, acc, idxs, idxd, rows, gsem, ssem) = refs

    c = lax.axis_index("c")
    s = lax.axis_index("s")
    r0 = s * RS

    @pl.when(s < 15)
    def _():
      pltpu.sync_copy(zrows, acc.at[pl.ds(r0, RS), :])

    @pl.when(s == 15)
    def _():
      pltpu.sync_copy(zrows.at[pl.ds(0, RS_LAST_A), :],
                      acc.at[pl.ds(15 * RS, RS_LAST_A), :])

    if with_deg:
      @pl.when(c == 0)
      def _():
        pltpu.sync_copy(ones_hbm, ones)

        @pl.when(s < 15)
        def _():
          pltpu.sync_copy(zflat.at[pl.ds(0, DZ)], dacc.at[pl.ds(s * DZ, DZ)])

        @pl.when(s == 15)
        def _():
          pltpu.sync_copy(zflat.at[pl.ds(0, DZ_LAST)],
                          dacc.at[pl.ds(15 * DZ, DZ_LAST)])
    plsc.subcore_barrier()

    def sup_body(t, carry):
      cid0 = s * CSUB + t * SUP
      pltpu.sync_copy(src2d.at[pl.ds(cid0, SUP), :], idxs)
      pltpu.sync_copy(dst2d.at[pl.ds(cid0, SUP), :], idxd)
      for half in range(SUP // 2):
        for j in range(2):
          q = 2 * half + j

          @pl.when(c == 0)
          def _():
            pltpu.async_copy(hlo.at[idxs.at[q]], rows.at[j], gsem)

          @pl.when(c == 1)
          def _():
            pltpu.async_copy(hhi.at[idxs.at[q]], rows.at[j], gsem)

        for j in range(2):
          q = 2 * half + j
          pltpu.make_async_copy(hlo.at[idxs.at[q]], rows.at[j], gsem).wait()
          pltpu.async_copy(rows.at[j], acc.at[idxd.at[q]], ssem, add=True)
          if with_deg:
            @pl.when(c == 0)
            def _():
              pltpu.sync_copy(ones, dacc.at[idxd.at[q]], add=True)
        for j in range(2):
          pltpu.make_async_copy(hlo.at[idxs.at[2 * half + j]], rows.at[j],
                                ssem).wait()
      return carry

    lax.fori_loop(0, NSUP, sup_body, 0)
    plsc.subcore_barrier()

    @pl.when(c == 0)
    def _():
      @pl.when(s < 15)
      def _():
        pltpu.sync_copy(acc.at[pl.ds(r0, RS), :], slo.at[pl.ds(r0, RS), :])

      @pl.when(s == 15)
      def _():
        pltpu.sync_copy(acc.at[pl.ds(15 * RS, RS_LAST_O), :],
                        slo.at[pl.ds(15 * RS, RS_LAST_O), :])

    @pl.when(c == 1)
    def _():
      @pl.when(s < 15)
      def _():
        pltpu.sync_copy(acc.at[pl.ds(r0, RS), :], shi.at[pl.ds(r0, RS), :])

      @pl.when(s == 15)
      def _():
        pltpu.sync_copy(acc.at[pl.ds(15 * RS, RS_LAST_O), :],
                        shi.at[pl.ds(15 * RS, RS_LAST_O), :])

    if with_deg:
      @pl.when((c == 0) & (s == 0))
      def _():
        pltpu.sync_copy(dacc, deg_hbm)

  return pl.kernel(body, out_type=out_type, mesh=mesh, scratch_types=scratch,
                   compiler_params=pltpu.CompilerParams(use_tc_tiling_on_sc=False),
                   name="seg_sum_deg" if with_deg else "seg_sum")


_seg_deg = _make_seg(True)
_seg = _make_seg(False)


def _full(shape):
  return pl.BlockSpec(shape, lambda i: tuple(0 for _ in shape))


def _rows(width):
  return pl.BlockSpec((BN, width), lambda i: (i, 0))


def _temb_body(t, wt1, bt1, wt2, bt2, wtime, bl, otb):
  tt = t[0, 0]
  idx = lax.broadcasted_iota(jnp.int32, (1, HALF), 1).astype(jnp.float32)
  freqs = jnp.exp(-jnp.log(10000.0) * idx / HALF)
  args = tt * freqs
  emb = jnp.concatenate([jnp.sin(args), jnp.cos(args)], axis=1)
  u = jnp.dot(emb, wt1[...], preferred_element_type=jnp.float32) + bt1[...]
  u = u * jax.nn.sigmoid(u)
  temb = jnp.dot(u, wt2[...], preferred_element_type=jnp.float32) + bt2[...]
  rows = [jnp.dot(temb, wtime[l], preferred_element_type=jnp.float32)
          + bl[l:l + 1, :] for l in range(4)]
  otb[...] = jnp.concatenate(rows, axis=0)


def _proj_body(lat, grad, w, b, olo, ohi):
  x = jnp.concatenate([lat[...], grad[...]], axis=1)
  h = jnp.dot(x, w[...], preferred_element_type=jnp.float32) + b[...]
  olo[...] = h[:, :HALF]
  ohi[...] = h[:, HALF:]


def _make_mid_body(with_skip):
  def body(*refs):
    if with_skip:
      hlo, hhi, slo, shi, deg, tb, wcat, sklo, skhi, olo, ohi = refs
    else:
      hlo, hhi, slo, shi, deg, tb, wcat, olo, ohi = refs
    h = jnp.concatenate([hlo[...], hhi[...]], axis=1)
    sn = jnp.concatenate([slo[...], shi[...]], axis=1)
    sn = sn / jnp.maximum(deg[...], 1.0)
    hs = jnp.concatenate([h, sn], axis=1)
    pre = jnp.dot(hs, wcat[...], preferred_element_type=jnp.float32) + tb[...]
    hn = jnp.maximum(pre, 0.0)
    if with_skip:
      hn = hn + jnp.concatenate([sklo[...], skhi[...]], axis=1)
    olo[...] = hn[:, :HALF]
    ohi[...] = hn[:, HALF:]
  return body


def _final_body(hlo, hhi, slo, shi, deg, tb, wcat, sklo, skhi,
                grad, wout, bout, out):
  h = jnp.concatenate([hlo[...], hhi[...]], axis=1)
  sn = jnp.concatenate([slo[...], shi[...]], axis=1)
  sn = sn / jnp.maximum(deg[...], 1.0)
  hs = jnp.concatenate([h, sn], axis=1)
  pre = jnp.dot(hs, wcat[...], preferred_element_type=jnp.float32) + tb[...]
  hn = jnp.maximum(pre, 0.0)
  hn = hn + jnp.concatenate([sklo[...], skhi[...]], axis=1)
  o = jnp.dot(hn, wout[...], preferred_element_type=jnp.float32) + bout[...]
  sg = jax.nn.sigmoid(o)
  out[...] = -grad[...] * sg[:, :HALF] + (sg[:, HALF:] * 2.0 - 1.0)


_half_out = [jax.ShapeDtypeStruct((N, HALF), jnp.float32)] * 2

_temb_call = pl.pallas_call(
    _temb_body,
    grid=(1,),
    in_specs=[_full((1, 1)), _full((TDIM, TDIM)), _full((1, TDIM)),
              _full((TDIM, HID)), _full((1, HID)), _full((4, HID, HID)),
              _full((4, HID))],
    out_specs=_full((4, HID)),
    out_shape=jax.ShapeDtypeStruct((4, HID), jnp.float32),
)

_proj_call = pl.pallas_call(
    _proj_body,
    grid=(GRID,),
    in_specs=[_rows(HALF), _rows(HALF), _full((HID, HID)), _full((1, HID))],
    out_specs=[_rows(HALF)] * 2,
    out_shape=_half_out,
)

_mid_specs = [_rows(HALF)] * 4 + [_rows(1), _full((1, HID)),
                                  _full((2 * HID, HID))]

_mid_call = pl.pallas_call(
    _make_mid_body(False),
    grid=(GRID,),
    in_specs=_mid_specs,
    out_specs=[_rows(HALF)] * 2,
    out_shape=_half_out,
)

_mid_skip_call = pl.pallas_call(
    _make_mid_body(True),
    grid=(GRID,),
    in_specs=_mid_specs + [_rows(HALF)] * 2,
    out_specs=[_rows(HALF)] * 2,
    out_shape=_half_out,
)

_final_call = pl.pallas_call(
    _final_body,
    grid=(GRID,),
    in_specs=_mid_specs + [_rows(HALF)] * 2
    + [_rows(HALF), _full((HID, HID)), _full((1, HID))],
    out_specs=_rows(HALF),
    out_shape=jax.ShapeDtypeStruct((N, HALF), jnp.float32),
)


def kernel(bxyz, latent, grad_latent, timestamp, edge_index, W_in, b_in,
           Wt1, bt1, Wt2, bt2, Wself, Wnbr, Wtime, bl, W_out, b_out):
  ei = edge_index.astype(jnp.int32)
  src2d = jnp.concatenate(
      [ei[0], jnp.zeros((EP - E,), jnp.int32)]).reshape(NCHUNK, CH)
  dst2d = jnp.concatenate(
      [ei[1], jnp.full((EP - E,), N, jnp.int32)]).reshape(NCHUNK, CH)
  zrows = jnp.zeros((RS, HALF), jnp.float32)
  zflat = jnp.zeros((DZ,), jnp.float32)
  ones_hbm = jnp.ones((CH,), jnp.float32)

  tb = _temb_call(timestamp.reshape(1, 1), Wt1, bt1.reshape(1, TDIM),
                  Wt2, bt2.reshape(1, HID), Wtime, bl)
  wcat = [jnp.concatenate([Wself[l], Wnbr[l]], axis=0) for l in range(4)]

  hlo, hhi = _proj_call(latent, grad_latent, W_in, b_in.reshape(1, HID))

  s0lo, s0hi, deg = _seg_deg(hlo, hhi, src2d, dst2d, zrows, zflat, ones_hbm)
  deg2d = deg[:N].reshape(N, 1)

  h1lo, h1hi = _mid_call(hlo, hhi, s0lo, s0hi, deg2d, tb[0:1], wcat[0])
  s1lo, s1hi = _seg(h1lo, h1hi, src2d, dst2d, zrows, zflat, ones_hbm)

  h2lo, h2hi = _mid_call(h1lo, h1hi, s1lo, s1hi, deg2d, tb[1:2], wcat[1])
  s2lo, s2hi = _seg(h2lo, h2hi, src2d, dst2d, zrows, zflat, ones_hbm)

  h3lo, h3hi = _mid_skip_call(h2lo, h2hi, s2lo, s2hi, deg2d, tb[2:3], wcat[2],
                              h2lo, h2hi)
  s3lo, s3hi = _seg(h3lo, h3hi, src2d, dst2d, zrows, zflat, ones_hbm)

  output = _final_call(h3lo, h3hi, s3lo, s3hi, deg2d, tb[3:4], wcat[3],
                       h1lo, h1hi, grad_latent, W_out, b_out.reshape(1, HID))
  return output, bxyz


# speedup vs baseline: 1.1106x; 1.0951x over previous
import functools

import jax
import jax.numpy as jnp
from jax import lax
from jax.experimental import pallas as pl
from jax.experimental.pallas import tpu as pltpu
from jax.experimental.pallas import tpu_sc as plsc

N = 50000
E = 800000
HID = 64
HALF = 32
TDIM = 64

NC = 2
NS = 16
CH = 128
SUP = 8
CSUB = 392
NCHUNK = CSUB * NS
EP = NCHUNK * CH
NSUP = CSUB // SUP

NA = N + 8
RS = 3128
RS_LAST_A = NA - 15 * RS
RS_LAST_O = N - 15 * RS

NDEG = NCHUNK * CH // 16
DZ = 3200
DZ_LAST = NDEG - 15 * DZ

BN = 2000
GRID = N // BN


def _make_seg(with_deg):
  mesh = plsc.VectorSubcoreMesh(
      core_axis_name="c", subcore_axis_name="s",
      num_cores=NC, num_subcores=NS)

  out_type = [jax.ShapeDtypeStruct((N, HALF), jnp.float32),
              jax.ShapeDtypeStruct((N, HALF), jnp.float32)]
  scratch = [
      pltpu.VMEM_SHARED((NA, HALF), jnp.float32),
      pltpu.VMEM((2, SUP, CH), jnp.int32),
      pltpu.VMEM((2, SUP, CH), jnp.int32),
      pltpu.VMEM((4, CH, HALF), jnp.float32),
      pltpu.SemaphoreType.DMA,
      pltpu.SemaphoreType.DMA,
      pltpu.SemaphoreType.DMA,
  ]
  if with_deg:
    out_type.append(jax.ShapeDtypeStruct((NDEG,), jnp.float32))
    scratch += [
        pltpu.VMEM_SHARED((NDEG,), jnp.float32),
        pltpu.VMEM((CH,), jnp.float32),
    ]

  def body(*refs):
    if with_deg:
      (hlo, hhi, src2d, dst2d, zrows, zflat, ones_hbm,
       slo, shi, deg_hbm, acc, idxs, idxd, rows, gsem, ssem, isem,
       dacc, ones) = refs
    else:
      (hlo, hhi, src2d, dst2d, zrows, zflat, ones_hbm,
       slo, shi, acc, idxs, idxd, rows, gsem, ssem, isem) = refs

    c = lax.axis_index("c")
    s = lax.axis_index("s")
    r0 = s * RS

    @pl.when(s < 15)
    def _():
      pltpu.sync_copy(zrows, acc.at[pl.ds(r0, RS), :])

    @pl.when(s == 15)
    def _():
      pltpu.sync_copy(zrows.at[pl.ds(0, RS_LAST_A), :],
                      acc.at[pl.ds(15 * RS, RS_LAST_A), :])

    if with_deg:
      @pl.when(c == 0)
      def _():
        pltpu.sync_copy(ones_hbm, ones)

        @pl.when(s < 15)
        def _():
          pltpu.sync_copy(zflat.at[pl.ds(0, DZ)], dacc.at[pl.ds(s * DZ, DZ)])

        @pl.when(s == 15)
        def _():
          pltpu.sync_copy(zflat.at[pl.ds(0, DZ_LAST)],
                          dacc.at[pl.ds(15 * DZ, DZ_LAST)])
    plsc.subcore_barrier()

    def drain_scatter(n):
      for _ in range(n):
        pltpu.make_async_copy(hlo.at[idxs.at[0, 0]], rows.at[0], ssem).wait()

    def sup_body(t, carry):
      p = t % 2
      cid0 = s * CSUB + t * SUP
      pltpu.async_copy(src2d.at[pl.ds(cid0, SUP), :], idxs.at[p], isem)
      pltpu.async_copy(dst2d.at[pl.ds(cid0, SUP), :], idxd.at[p], isem)

      @pl.when(t > 0)
      def _():
        drain_scatter(4)

      pltpu.make_async_copy(src2d.at[pl.ds(cid0, SUP), :], idxs.at[p],
                            isem).wait()
      pltpu.make_async_copy(dst2d.at[pl.ds(cid0, SUP), :], idxd.at[p],
                            isem).wait()

      for b in range(2):
        for j in range(4):
          q = 4 * b + j

          @pl.when(c == 0)
          def _():
            pltpu.async_copy(hlo.at[idxs.at[p, q]], rows.at[j], gsem)

          @pl.when(c == 1)
          def _():
            pltpu.async_copy(hhi.at[idxs.at[p, q]], rows.at[j], gsem)

        for j in range(4):
          q = 4 * b + j
          pltpu.make_async_copy(hlo.at[idxs.at[p, q]], rows.at[j],
                                gsem).wait()
          pltpu.async_copy(rows.at[j], acc.at[idxd.at[p, q]], ssem, add=True)
          if with_deg:
            @pl.when(c == 0)
            def _():
              pltpu.sync_copy(ones, dacc.at[idxd.at[p, q]], add=True)
        if b == 0:
          drain_scatter(4)
      return carry

    lax.fori_loop(0, NSUP, sup_body, 0)
    drain_scatter(4)
    plsc.subcore_barrier()

    @pl.when(c == 0)
    def _():
      @pl.when(s < 15)
      def _():
        pltpu.sync_copy(acc.at[pl.ds(r0, RS), :], slo.at[pl.ds(r0, RS), :])

      @pl.when(s == 15)
      def _():
        pltpu.sync_copy(acc.at[pl.ds(15 * RS, RS_LAST_O), :],
                        slo.at[pl.ds(15 * RS, RS_LAST_O), :])

    @pl.when(c == 1)
    def _():
      @pl.when(s < 15)
      def _():
        pltpu.sync_copy(acc.at[pl.ds(r0, RS), :], shi.at[pl.ds(r0, RS), :])

      @pl.when(s == 15)
      def _():
        pltpu.sync_copy(acc.at[pl.ds(15 * RS, RS_LAST_O), :],
                        shi.at[pl.ds(15 * RS, RS_LAST_O), :])

    if with_deg:
      @pl.when((c == 0) & (s == 0))
      def _():
        pltpu.sync_copy(dacc, deg_hbm)

  return pl.kernel(body, out_type=out_type, mesh=mesh, scratch_types=scratch,
                   compiler_params=pltpu.CompilerParams(use_tc_tiling_on_sc=False),
                   name="seg_sum_deg" if with_deg else "seg_sum")


_seg_deg = _make_seg(True)
_seg = _make_seg(False)


def _full(shape):
  return pl.BlockSpec(shape, lambda i: tuple(0 for _ in shape))


def _rows(width):
  return pl.BlockSpec((BN, width), lambda i: (i, 0))


def _temb_body(t, wt1, bt1, wt2, bt2, wtime, bl, otb):
  tt = t[0, 0]
  idx = lax.broadcasted_iota(jnp.int32, (1, HALF), 1).astype(jnp.float32)
  freqs = jnp.exp(-jnp.log(10000.0) * idx / HALF)
  args = tt * freqs
  emb = jnp.concatenate([jnp.sin(args), jnp.cos(args)], axis=1)
  u = jnp.dot(emb, wt1[...], preferred_element_type=jnp.float32) + bt1[...]
  u = u * jax.nn.sigmoid(u)
  temb = jnp.dot(u, wt2[...], preferred_element_type=jnp.float32) + bt2[...]
  rows = [jnp.dot(temb, wtime[l], preferred_element_type=jnp.float32)
          + bl[l:l + 1, :] for l in range(4)]
  otb[...] = jnp.concatenate(rows, axis=0)


def _proj_body(lat, grad, w, b, olo, ohi):
  x = jnp.concatenate([lat[...], grad[...]], axis=1)
  h = jnp.dot(x, w[...], preferred_element_type=jnp.float32) + b[...]
  olo[...] = h[:, :HALF]
  ohi[...] = h[:, HALF:]


def _make_mid_body(with_skip):
  def body(*refs):
    if with_skip:
      hlo, hhi, slo, shi, deg, tb, wcat, sklo, skhi, olo, ohi = refs
    else:
      hlo, hhi, slo, shi, deg, tb, wcat, olo, ohi = refs
    h = jnp.concatenate([hlo[...], hhi[...]], axis=1)
    sn = jnp.concatenate([slo[...], shi[...]], axis=1)
    sn = sn / jnp.maximum(deg[...], 1.0)
    hs = jnp.concatenate([h, sn], axis=1)
    pre = jnp.dot(hs, wcat[...], preferred_element_type=jnp.float32) + tb[...]
    hn = jnp.maximum(pre, 0.0)
    if with_skip:
      hn = hn + jnp.concatenate([sklo[...], skhi[...]], axis=1)
    olo[...] = hn[:, :HALF]
    ohi[...] = hn[:, HALF:]
  return body


def _final_body(hlo, hhi, slo, shi, deg, tb, wcat, sklo, skhi,
                grad, wout, bout, out):
  h = jnp.concatenate([hlo[...], hhi[...]], axis=1)
  sn = jnp.concatenate([slo[...], shi[...]], axis=1)
  sn = sn / jnp.maximum(deg[...], 1.0)
  hs = jnp.concatenate([h, sn], axis=1)
  pre = jnp.dot(hs, wcat[...], preferred_element_type=jnp.float32) + tb[...]
  hn = jnp.maximum(pre, 0.0)
  hn = hn + jnp.concatenate([sklo[...], skhi[...]], axis=1)
  o = jnp.dot(hn, wout[...], preferred_element_type=jnp.float32) + bout[...]
  sg = jax.nn.sigmoid(o)
  out[...] = -grad[...] * sg[:, :HALF] + (sg[:, HALF:] * 2.0 - 1.0)


_half_out = [jax.ShapeDtypeStruct((N, HALF), jnp.float32)] * 2

_temb_call = pl.pallas_call(
    _temb_body,
    grid=(1,),
    in_specs=[_full((1, 1)), _full((TDIM, TDIM)), _full((1, TDIM)),
              _full((TDIM, HID)), _full((1, HID)), _full((4, HID, HID)),
              _full((4, HID))],
    out_specs=_full((4, HID)),
    out_shape=jax.ShapeDtypeStruct((4, HID), jnp.float32),
)

_proj_call = pl.pallas_call(
    _proj_body,
    grid=(GRID,),
    in_specs=[_rows(HALF), _rows(HALF), _full((HID, HID)), _full((1, HID))],
    out_specs=[_rows(HALF)] * 2,
    out_shape=_half_out,
)

_mid_specs = [_rows(HALF)] * 4 + [_rows(1), _full((1, HID)),
                                  _full((2 * HID, HID))]

_mid_call = pl.pallas_call(
    _make_mid_body(False),
    grid=(GRID,),
    in_specs=_mid_specs,
    out_specs=[_rows(HALF)] * 2,
    out_shape=_half_out,
)

_mid_skip_call = pl.pallas_call(
    _make_mid_body(True),
    grid=(GRID,),
    in_specs=_mid_specs + [_rows(HALF)] * 2,
    out_specs=[_rows(HALF)] * 2,
    out_shape=_half_out,
)

_final_call = pl.pallas_call(
    _final_body,
    grid=(GRID,),
    in_specs=_mid_specs + [_rows(HALF)] * 2
    + [_rows(HALF), _full((HID, HID)), _full((1, HID))],
    out_specs=_rows(HALF),
    out_shape=jax.ShapeDtypeStruct((N, HALF), jnp.float32),
)


def kernel(bxyz, latent, grad_latent, timestamp, edge_index, W_in, b_in,
           Wt1, bt1, Wt2, bt2, Wself, Wnbr, Wtime, bl, W_out, b_out):
  ei = edge_index.astype(jnp.int32)
  src2d = jnp.concatenate(
      [ei[0], jnp.zeros((EP - E,), jnp.int32)]).reshape(NCHUNK, CH)
  dst2d = jnp.concatenate(
      [ei[1], jnp.full((EP - E,), N, jnp.int32)]).reshape(NCHUNK, CH)
  zrows = jnp.zeros((RS, HALF), jnp.float32)
  zflat = jnp.zeros((DZ,), jnp.float32)
  ones_hbm = jnp.ones((CH,), jnp.float32)

  tb = _temb_call(timestamp.reshape(1, 1), Wt1, bt1.reshape(1, TDIM),
                  Wt2, bt2.reshape(1, HID), Wtime, bl)
  wcat = [jnp.concatenate([Wself[l], Wnbr[l]], axis=0) for l in range(4)]

  hlo, hhi = _proj_call(latent, grad_latent, W_in, b_in.reshape(1, HID))

  s0lo, s0hi, deg = _seg_deg(hlo, hhi, src2d, dst2d, zrows, zflat, ones_hbm)
  deg2d = deg[:N].reshape(N, 1)

  h1lo, h1hi = _mid_call(hlo, hhi, s0lo, s0hi, deg2d, tb[0:1], wcat[0])
  s1lo, s1hi = _seg(h1lo, h1hi, src2d, dst2d, zrows, zflat, ones_hbm)

  h2lo, h2hi = _mid_call(h1lo, h1hi, s1lo, s1hi, deg2d, tb[1:2], wcat[1])
  s2lo, s2hi = _seg(h2lo, h2hi, src2d, dst2d, zrows, zflat, ones_hbm)

  h3lo, h3hi = _mid_skip_call(h2lo, h2hi, s2lo, s2hi, deg2d, tb[2:3], wcat[2],
                              h2lo, h2hi)
  s3lo, s3hi = _seg(h3lo, h3hi, src2d, dst2d, zrows, zflat, ones_hbm)

  output = _final_call(h3lo, h3hi, s3lo, s3hi, deg2d, tb[3:4], wcat[3],
                       h1lo, h1hi, grad_latent, W_out, b_out.reshape(1, HID))
  return output, bxyz


# speedup vs baseline: 1.1274x; 1.0152x over previous
import functools

import jax
import jax.numpy as jnp
from jax import lax
from jax.experimental import pallas as pl
from jax.experimental.pallas import tpu as pltpu
from jax.experimental.pallas import tpu_sc as plsc

N = 50000
E = 800000
HID = 64
HALF = 32
TDIM = 64

NC = 2
NS = 16
CH = 128
SUP = 8
CSUB = 392
NCHUNK = CSUB * NS
EP = NCHUNK * CH
NSUP = CSUB // SUP

NA = N + 8
RS = 3128
RS_LAST_A = NA - 15 * RS
RS_LAST_O = N - 15 * RS

NDEG = NCHUNK * CH // 16
DZ = 3200
DZ_LAST = NDEG - 15 * DZ

BN = 5000
GRID = N // BN


def _make_seg(with_deg):
  mesh = plsc.VectorSubcoreMesh(
      core_axis_name="c", subcore_axis_name="s",
      num_cores=NC, num_subcores=NS)

  out_type = [jax.ShapeDtypeStruct((N, HALF), jnp.float32),
              jax.ShapeDtypeStruct((N, HALF), jnp.float32)]
  scratch = [
      pltpu.VMEM_SHARED((NA, HALF), jnp.float32),
      pltpu.VMEM((2, SUP, CH), jnp.int32),
      pltpu.VMEM((2, SUP, CH), jnp.int32),
      pltpu.VMEM((4, CH, HALF), jnp.float32),
      pltpu.SemaphoreType.DMA,
      pltpu.SemaphoreType.DMA,
      pltpu.SemaphoreType.DMA,
  ]
  if with_deg:
    out_type.append(jax.ShapeDtypeStruct((NDEG,), jnp.float32))
    scratch += [
        pltpu.VMEM_SHARED((NDEG,), jnp.float32),
        pltpu.VMEM((CH,), jnp.float32),
    ]

  def body(*refs):
    if with_deg:
      (hlo, hhi, src2d, dst2d, zrows, zflat, ones_hbm,
       slo, shi, deg_hbm, acc, idxs, idxd, rows, gsem, ssem, isem,
       dacc, ones) = refs
    else:
      (hlo, hhi, src2d, dst2d, zrows, zflat, ones_hbm,
       slo, shi, acc, idxs, idxd, rows, gsem, ssem, isem) = refs

    c = lax.axis_index("c")
    s = lax.axis_index("s")
    r0 = s * RS

    @pl.when(s < 15)
    def _():
      pltpu.sync_copy(zrows, acc.at[pl.ds(r0, RS), :])

    @pl.when(s == 15)
    def _():
      pltpu.sync_copy(zrows.at[pl.ds(0, RS_LAST_A), :],
                      acc.at[pl.ds(15 * RS, RS_LAST_A), :])

    if with_deg:
      @pl.when(c == 0)
      def _():
        pltpu.sync_copy(ones_hbm, ones)

        @pl.when(s < 15)
        def _():
          pltpu.sync_copy(zflat.at[pl.ds(0, DZ)], dacc.at[pl.ds(s * DZ, DZ)])

        @pl.when(s == 15)
        def _():
          pltpu.sync_copy(zflat.at[pl.ds(0, DZ_LAST)],
                          dacc.at[pl.ds(15 * DZ, DZ_LAST)])
    plsc.subcore_barrier()

    def drain_scatter(n):
      for _ in range(n):
        pltpu.make_async_copy(hlo.at[idxs.at[0, 0]], rows.at[0], ssem).wait()

    def sup_body(t, carry):
      p = t % 2
      cid0 = s * CSUB + t * SUP
      pltpu.async_copy(src2d.at[pl.ds(cid0, SUP), :], idxs.at[p], isem)
      pltpu.async_copy(dst2d.at[pl.ds(cid0, SUP), :], idxd.at[p], isem)

      @pl.when(t > 0)
      def _():
        drain_scatter(4)

      pltpu.make_async_copy(src2d.at[pl.ds(cid0, SUP), :], idxs.at[p],
                            isem).wait()
      pltpu.make_async_copy(dst2d.at[pl.ds(cid0, SUP), :], idxd.at[p],
                            isem).wait()

      for b in range(2):
        for j in range(4):
          q = 4 * b + j

          @pl.when(c == 0)
          def _():
            pltpu.async_copy(hlo.at[idxs.at[p, q]], rows.at[j], gsem)

          @pl.when(c == 1)
          def _():
            pltpu.async_copy(hhi.at[idxs.at[p, q]], rows.at[j], gsem)

        for j in range(4):
          q = 4 * b + j
          pltpu.make_async_copy(hlo.at[idxs.at[p, q]], rows.at[j],
                                gsem).wait()
          pltpu.async_copy(rows.at[j], acc.at[idxd.at[p, q]], ssem, add=True)
          if with_deg:
            @pl.when(c == 0)
            def _():
              pltpu.sync_copy(ones, dacc.at[idxd.at[p, q]], add=True)
        if b == 0:
          drain_scatter(4)
      return carry

    lax.fori_loop(0, NSUP, sup_body, 0)
    drain_scatter(4)
    plsc.subcore_barrier()

    @pl.when(c == 0)
    def _():
      @pl.when(s < 15)
      def _():
        pltpu.sync_copy(acc.at[pl.ds(r0, RS), :], slo.at[pl.ds(r0, RS), :])

      @pl.when(s == 15)
      def _():
        pltpu.sync_copy(acc.at[pl.ds(15 * RS, RS_LAST_O), :],
                        slo.at[pl.ds(15 * RS, RS_LAST_O), :])

    @pl.when(c == 1)
    def _():
      @pl.when(s < 15)
      def _():
        pltpu.sync_copy(acc.at[pl.ds(r0, RS), :], shi.at[pl.ds(r0, RS), :])

      @pl.when(s == 15)
      def _():
        pltpu.sync_copy(acc.at[pl.ds(15 * RS, RS_LAST_O), :],
                        shi.at[pl.ds(15 * RS, RS_LAST_O), :])

    if with_deg:
      @pl.when((c == 0) & (s == 0))
      def _():
        pltpu.sync_copy(dacc, deg_hbm)

  return pl.kernel(body, out_type=out_type, mesh=mesh, scratch_types=scratch,
                   compiler_params=pltpu.CompilerParams(use_tc_tiling_on_sc=False),
                   name="seg_sum_deg" if with_deg else "seg_sum")


_seg_deg = _make_seg(True)
_seg = _make_seg(False)


def _full(shape):
  return pl.BlockSpec(shape, lambda i: tuple(0 for _ in shape))


def _rows(width):
  return pl.BlockSpec((BN, width), lambda i: (i, 0))


def _temb_body(t, wt1, bt1, wt2, bt2, wtime, bl, otb):
  tt = t[0, 0]
  idx = lax.broadcasted_iota(jnp.int32, (1, HALF), 1).astype(jnp.float32)
  freqs = jnp.exp(-jnp.log(10000.0) * idx / HALF)
  args = tt * freqs
  emb = jnp.concatenate([jnp.sin(args), jnp.cos(args)], axis=1)
  u = jnp.dot(emb, wt1[...], preferred_element_type=jnp.float32) + bt1[...]
  u = u * jax.nn.sigmoid(u)
  temb = jnp.dot(u, wt2[...], preferred_element_type=jnp.float32) + bt2[...]
  rows = [jnp.dot(temb, wtime[l], preferred_element_type=jnp.float32)
          + bl[l:l + 1, :] for l in range(4)]
  otb[...] = jnp.concatenate(rows, axis=0)


def _invdeg_body(deg, out):
  d = jnp.maximum(deg[...], 1.0)
  out[...] = jnp.broadcast_to(1.0 / d, (BN, HALF))


def _proj_body(lat, grad, w, b, olo, ohi):
  x = jnp.concatenate([lat[...], grad[...]], axis=1)
  h = jnp.dot(x, w[...], preferred_element_type=jnp.float32) + b[...]
  olo[...] = h[:, :HALF]
  ohi[...] = h[:, HALF:]


def _make_mid_body(with_skip):
  def body(*refs):
    if with_skip:
      hlo, hhi, slo, shi, deg, tb, wcat, sklo, skhi, olo, ohi = refs
    else:
      hlo, hhi, slo, shi, deg, tb, wcat, olo, ohi = refs
    h = jnp.concatenate([hlo[...], hhi[...]], axis=1)
    iv = deg[...]
    hs = jnp.concatenate([h, slo[...] * iv, shi[...] * iv], axis=1)
    pre = jnp.dot(hs, wcat[...], preferred_element_type=jnp.float32) + tb[...]
    hn = jnp.maximum(pre, 0.0)
    if with_skip:
      hn = hn + jnp.concatenate([sklo[...], skhi[...]], axis=1)
    olo[...] = hn[:, :HALF]
    ohi[...] = hn[:, HALF:]
  return body


def _final_body(hlo, hhi, slo, shi, deg, tb, wcat, sklo, skhi,
                grad, wout, bout, out):
  h = jnp.concatenate([hlo[...], hhi[...]], axis=1)
  iv = deg[...]
  hs = jnp.concatenate([h, slo[...] * iv, shi[...] * iv], axis=1)
  pre = jnp.dot(hs, wcat[...], preferred_element_type=jnp.float32) + tb[...]
  hn = jnp.maximum(pre, 0.0)
  hn = hn + jnp.concatenate([sklo[...], skhi[...]], axis=1)
  o = jnp.dot(hn, wout[...], preferred_element_type=jnp.float32) + bout[...]
  sg = jax.nn.sigmoid(o)
  out[...] = -grad[...] * sg[:, :HALF] + (sg[:, HALF:] * 2.0 - 1.0)


_half_out = [jax.ShapeDtypeStruct((N, HALF), jnp.float32)] * 2

_temb_call = pl.pallas_call(
    _temb_body,
    grid=(1,),
    in_specs=[_full((1, 1)), _full((TDIM, TDIM)), _full((1, TDIM)),
              _full((TDIM, HID)), _full((1, HID)), _full((4, HID, HID)),
              _full((4, HID))],
    out_specs=_full((4, HID)),
    out_shape=jax.ShapeDtypeStruct((4, HID), jnp.float32),
)

_proj_call = pl.pallas_call(
    _proj_body,
    grid=(GRID,),
    in_specs=[_rows(HALF), _rows(HALF), _full((HID, HID)), _full((1, HID))],
    out_specs=[_rows(HALF)] * 2,
    out_shape=_half_out,
)

_invdeg_call = pl.pallas_call(
    _invdeg_body,
    grid=(GRID,),
    in_specs=[_rows(1)],
    out_specs=_rows(HALF),
    out_shape=jax.ShapeDtypeStruct((N, HALF), jnp.float32),
)

_mid_specs = [_rows(HALF)] * 4 + [_rows(HALF), _full((1, HID)),
                                  _full((2 * HID, HID))]

_mid_call = pl.pallas_call(
    _make_mid_body(False),
    grid=(GRID,),
    in_specs=_mid_specs,
    out_specs=[_rows(HALF)] * 2,
    out_shape=_half_out,
)

_mid_skip_call = pl.pallas_call(
    _make_mid_body(True),
    grid=(GRID,),
    in_specs=_mid_specs + [_rows(HALF)] * 2,
    out_specs=[_rows(HALF)] * 2,
    out_shape=_half_out,
)

_final_call = pl.pallas_call(
    _final_body,
    grid=(GRID,),
    in_specs=_mid_specs + [_rows(HALF)] * 2
    + [_rows(HALF), _full((HID, HID)), _full((1, HID))],
    out_specs=_rows(HALF),
    out_shape=jax.ShapeDtypeStruct((N, HALF), jnp.float32),
)


def kernel(bxyz, latent, grad_latent, timestamp, edge_index, W_in, b_in,
           Wt1, bt1, Wt2, bt2, Wself, Wnbr, Wtime, bl, W_out, b_out):
  ei = edge_index.astype(jnp.int32)
  src2d = jnp.concatenate(
      [ei[0], jnp.zeros((EP - E,), jnp.int32)]).reshape(NCHUNK, CH)
  dst2d = jnp.concatenate(
      [ei[1], jnp.full((EP - E,), N, jnp.int32)]).reshape(NCHUNK, CH)
  zrows = jnp.zeros((RS, HALF), jnp.float32)
  zflat = jnp.zeros((DZ,), jnp.float32)
  ones_hbm = jnp.ones((CH,), jnp.float32)

  tb = _temb_call(timestamp.reshape(1, 1), Wt1, bt1.reshape(1, TDIM),
                  Wt2, bt2.reshape(1, HID), Wtime, bl)
  wcat = [jnp.concatenate([Wself[l], Wnbr[l]], axis=0) for l in range(4)]

  hlo, hhi = _proj_call(latent, grad_latent, W_in, b_in.reshape(1, HID))

  s0lo, s0hi, deg = _seg_deg(hlo, hhi, src2d, dst2d, zrows, zflat, ones_hbm)
  deg2d = _invdeg_call(deg[:N].reshape(N, 1))

  h1lo, h1hi = _mid_call(hlo, hhi, s0lo, s0hi, deg2d, tb[0:1], wcat[0])
  s1lo, s1hi = _seg(h1lo, h1hi, src2d, dst2d, zrows, zflat, ones_hbm)

  h2lo, h2hi = _mid_call(h1lo, h1hi, s1lo, s1hi, deg2d, tb[1:2], wcat[1])
  s2lo, s2hi = _seg(h2lo, h2hi, src2d, dst2d, zrows, zflat, ones_hbm)

  h3lo, h3hi = _mid_skip_call(h2lo, h2hi, s2lo, s2hi, deg2d, tb[2:3], wcat[2],
                              h2lo, h2hi)
  s3lo, s3hi = _seg(h3lo, h3hi, src2d, dst2d, zrows, zflat, ones_hbm)

  output = _final_call(h3lo, h3hi, s3lo, s3hi, deg2d, tb[3:4], wcat[3],
                       h1lo, h1hi, grad_latent, W_out, b_out.reshape(1, HID))
  return output, bxyz
